# Initial kernel scaffold; baseline (speedup 1.0000x reference)
import jax
import jax.numpy as jnp
from jax.experimental import pallas as pl


def kernel(x, edge_index, edge_weight, W1, W2):
    raise NotImplementedError("write your pallas kernel here")


# speedup vs baseline: 2.8595x; 2.8595x over previous
import jax
import jax.numpy as jnp
from jax import lax
from jax.experimental import pallas as pl
from jax.experimental.pallas import tpu as pltpu
from jax.experimental.pallas import tpu_sc as plsc

N = 10000
NP = 10240
E = 320000
D = 128
H1 = 128
H2 = 64

NC = 2
NS = 16
CB = 80
ROWS_PER_TEC = NP // NS
PER_TEC = E // NC // NS
N_CHUNKS = PER_TEC // CB


def _mm1_body(x_ref, w_ref, o_ref):
    o_ref[...] = jnp.dot(x_ref[...], w_ref[...],
                         preferred_element_type=jnp.float32)


def _mm1(x, W1):
    R = 1000
    return pl.pallas_call(
        _mm1_body,
        grid=(N // R,),
        in_specs=[
            pl.BlockSpec((R, D), lambda i: (i, 0)),
            pl.BlockSpec((D, H1), lambda i: (0, 0)),
        ],
        out_specs=pl.BlockSpec((R, H1), lambda i: (i, 0)),
        out_shape=jax.ShapeDtypeStruct((NP, H1), jnp.float32),
    )(x, W1)


def _mm2_body(p_ref, w2_ref, o_ref):
    a = jnp.maximum(p_ref[0] + p_ref[1], 0.0)
    h2 = jnp.dot(a, w2_ref[...], preferred_element_type=jnp.float32)
    o_ref[...] = jnp.concatenate([h2, jnp.zeros_like(h2)], axis=1)


def _mm2(parts, W2):
    R = 1000
    return pl.pallas_call(
        _mm2_body,
        grid=(N // R,),
        in_specs=[
            pl.BlockSpec((2, R, H1), lambda i: (0, i, 0)),
            pl.BlockSpec((H1, H2), lambda i: (0, 0)),
        ],
        out_specs=pl.BlockSpec((R, H1), lambda i: (i, 0)),
        out_shape=jax.ShapeDtypeStruct((NP, H1), jnp.float32),
    )(parts, W2)


def _softmax_body(p_ref, o_ref):
    s = p_ref[0, :, :H2] + p_ref[1, :, :H2]
    m = jnp.max(s, axis=-1, keepdims=True)
    e = jnp.exp(s - m)
    o_ref[...] = e / jnp.sum(e, axis=-1, keepdims=True)


def _softmax(parts):
    R = 1000
    return pl.pallas_call(
        _softmax_body,
        grid=(N // R,),
        in_specs=[pl.BlockSpec((2, R, H1), lambda i: (0, i, 0))],
        out_specs=pl.BlockSpec((R, H2), lambda i: (i, 0)),
        out_shape=jax.ShapeDtypeStruct((N, H2), jnp.float32),
    )(parts)


def _sc_body(table_hbm, src_hbm, dst_hbm, w_hbm, zeros_hbm, out_hbm,
             agg_sh, sidx, didx, wbuf, rows, gsem):
    c = lax.axis_index("c")
    s = lax.axis_index("s")
    rsl = pl.ds(s * ROWS_PER_TEC, ROWS_PER_TEC)
    pltpu.sync_copy(zeros_hbm.at[rsl], agg_sh.at[rsl])
    plsc.subcore_barrier()

    base0 = c * (E // NC) + s * PER_TEC

    def chunk(i, carry):
        b = base0 + i * CB
        pltpu.sync_copy(src_hbm.at[pl.ds(b, CB)], sidx.at[0])
        pltpu.sync_copy(dst_hbm.at[pl.ds(b, CB)], didx.at[0])
        pltpu.sync_copy(w_hbm.at[pl.ds(b, CB)], wbuf)
        pltpu.async_copy(table_hbm.at[sidx.at[0]], rows, gsem).wait()

        def scale(e, c2):
            for j in range(H1 // 16):
                sl = pl.ds(j * 16, 16)
                rows[e, sl] = rows[e, sl] * wbuf[e, sl]
            return c2

        lax.fori_loop(0, CB, scale, 0)
        pltpu.sync_copy(rows, agg_sh.at[didx.at[0]], add=True)
        return carry

    lax.fori_loop(0, N_CHUNKS, chunk, 0)
    plsc.subcore_barrier()
    pltpu.sync_copy(agg_sh.at[rsl], out_hbm.at[c, rsl])


_sc_pass = pl.kernel(
    _sc_body,
    out_type=jax.ShapeDtypeStruct((2, NP, H1), jnp.float32),
    mesh=plsc.VectorSubcoreMesh(core_axis_name="c", subcore_axis_name="s",
                                num_cores=NC, num_subcores=NS),
    scratch_types=[
        pltpu.VMEM_SHARED((NP, H1), jnp.float32),
        pltpu.VMEM((1, CB), jnp.int32),
        pltpu.VMEM((1, CB), jnp.int32),
        pltpu.VMEM((CB, H1), jnp.float32),
        pltpu.VMEM((CB, H1), jnp.float32),
        pltpu.SemaphoreType.DMA,
    ],
)


@jax.jit
def kernel(x, edge_index, edge_weight, W1, W2):
    src = edge_index[0].astype(jnp.int32)
    dst = edge_index[1].astype(jnp.int32)
    w = jnp.broadcast_to(edge_weight.astype(jnp.float32)[:, None], (E, H1))
    zeros = jnp.zeros((NP, H1), jnp.float32)

    h = _mm1(x, W1)
    p1 = _sc_pass(h, src, dst, w, zeros)
    h2 = _mm2(p1, W2)
    p2 = _sc_pass(h2, src, dst, w, zeros)
    return _softmax(p2)


# speedup vs baseline: 3.0458x; 1.0652x over previous
import jax
import jax.numpy as jnp
from jax import lax
from jax.experimental import pallas as pl
from jax.experimental.pallas import tpu as pltpu
from jax.experimental.pallas import tpu_sc as plsc

N = 10000
NP = 10240
E = 320000
D = 128
H1 = 128
H2 = 64

NC = 2
NS = 16
CB = 80
ROWS_PER_TEC = NP // NS
PER_TEC = E // NC // NS
N_CHUNKS = PER_TEC // CB


def _mm1_body(x_ref, w_ref, o_ref):
    o_ref[...] = jnp.dot(x_ref[...], w_ref[...],
                         preferred_element_type=jnp.float32)


def _mm1(x, W1):
    R = 1000
    return pl.pallas_call(
        _mm1_body,
        grid=(N // R,),
        in_specs=[
            pl.BlockSpec((R, D), lambda i: (i, 0)),
            pl.BlockSpec((D, H1), lambda i: (0, 0)),
        ],
        out_specs=pl.BlockSpec((R, H1), lambda i: (i, 0)),
        out_shape=jax.ShapeDtypeStruct((NP, H1), jnp.float32),
    )(x, W1)


def _mm2_body(p_ref, w2_ref, o_ref):
    a = jnp.maximum(p_ref[0] + p_ref[1], 0.0)
    h2 = jnp.dot(a, w2_ref[...], preferred_element_type=jnp.float32)
    o_ref[...] = jnp.concatenate([h2, jnp.zeros_like(h2)], axis=1)


def _mm2(parts, W2):
    R = 1000
    return pl.pallas_call(
        _mm2_body,
        grid=(N // R,),
        in_specs=[
            pl.BlockSpec((2, R, H1), lambda i: (0, i, 0)),
            pl.BlockSpec((H1, H2), lambda i: (0, 0)),
        ],
        out_specs=pl.BlockSpec((R, H1), lambda i: (i, 0)),
        out_shape=jax.ShapeDtypeStruct((NP, H1), jnp.float32),
    )(parts, W2)


def _softmax_body(p_ref, o_ref):
    s = p_ref[0, :, :H2] + p_ref[1, :, :H2]
    m = jnp.max(s, axis=-1, keepdims=True)
    e = jnp.exp(s - m)
    o_ref[...] = e / jnp.sum(e, axis=-1, keepdims=True)


def _softmax(parts):
    R = 1000
    return pl.pallas_call(
        _softmax_body,
        grid=(N // R,),
        in_specs=[pl.BlockSpec((2, R, H1), lambda i: (0, i, 0))],
        out_specs=pl.BlockSpec((R, H2), lambda i: (i, 0)),
        out_shape=jax.ShapeDtypeStruct((N, H2), jnp.float32),
    )(parts)


def _sc_body(table_hbm, src_hbm, dst_hbm, w_hbm, zeros_hbm, out_hbm,
             agg_sh, sidx, didx, wbuf, rows, gsem):
    c = lax.axis_index("c")
    s = lax.axis_index("s")
    rsl = pl.ds(s * ROWS_PER_TEC, ROWS_PER_TEC)
    pltpu.sync_copy(zeros_hbm.at[rsl], agg_sh.at[rsl])
    plsc.subcore_barrier()

    base0 = c * (E // NC) + s * PER_TEC

    def chunk(i, carry):
        b = base0 + i * CB
        pltpu.sync_copy(src_hbm.at[pl.ds(b, CB)], sidx.at[0])
        pltpu.sync_copy(dst_hbm.at[pl.ds(b, CB)], didx.at[0])
        pltpu.sync_copy(w_hbm.at[pl.ds(b, CB)], wbuf)
        pltpu.async_copy(table_hbm.at[sidx.at[0]], rows, gsem).wait()

        def scale(e, c2):
            we = wbuf[e, pl.ds(0, 16)]
            for j in range(H1 // 16):
                sl = pl.ds(j * 16, 16)
                rows[e, sl] = rows[e, sl] * we
            return c2

        lax.fori_loop(0, CB, scale, 0)
        pltpu.sync_copy(rows, agg_sh.at[didx.at[0]], add=True)
        return carry

    lax.fori_loop(0, N_CHUNKS, chunk, 0)
    plsc.subcore_barrier()
    pltpu.sync_copy(agg_sh.at[rsl], out_hbm.at[c, rsl])


_sc_pass = pl.kernel(
    _sc_body,
    out_type=jax.ShapeDtypeStruct((2, NP, H1), jnp.float32),
    mesh=plsc.VectorSubcoreMesh(core_axis_name="c", subcore_axis_name="s",
                                num_cores=NC, num_subcores=NS),
    scratch_types=[
        pltpu.VMEM_SHARED((NP, H1), jnp.float32),
        pltpu.VMEM((1, CB), jnp.int32),
        pltpu.VMEM((1, CB), jnp.int32),
        pltpu.VMEM((CB, H1), jnp.float32),
        pltpu.VMEM((CB, H1), jnp.float32),
        pltpu.SemaphoreType.DMA,
    ],
)


@jax.jit
def kernel(x, edge_index, edge_weight, W1, W2):
    src = edge_index[0].astype(jnp.int32)
    dst = edge_index[1].astype(jnp.int32)
    w = jnp.broadcast_to(edge_weight.astype(jnp.float32)[:, None], (E, H1))
    zeros = jnp.zeros((NP, H1), jnp.float32)

    h = _mm1(x, W1)
    p1 = _sc_pass(h, src, dst, w, zeros)
    h2 = _mm2(p1, W2)
    p2 = _sc_pass(h2, src, dst, w, zeros)
    return _softmax(p2)
